# baseline (speedup 1.0000x reference)
import functools

import jax
import jax.numpy as jnp
from jax import lax
from jax.experimental import pallas as pl
from jax.experimental.pallas import tpu as pltpu
from jax.experimental.pallas import tpu_sc as plsc

B, N, K = 8, 2048, 16
M = B * N
TR = 1024
NW = 32
GCH = 128 // K
BG = 2
NBUF = 2


def _topk_body(ct_ref, ca_ref, x_ref, w_ref, b_ref, idx_ref, p_ref, q_ref):
    b = pl.program_id(0)
    h = jnp.dot(x_ref[...], w_ref[...],
                preferred_element_type=jnp.float32) + b_ref[...]
    p_ref[...] = h[:, :64]
    q_ref[...] = h[:, 64:]
    ct = ct_ref[0]
    ca = ca_ref[0]
    dot = jnp.dot(ct, ca, preferred_element_type=jnp.float32)
    sqt = jnp.sum(ct * ct, axis=1)
    sqa = jnp.sum(ca * ca, axis=0)
    x = 2.0 * dot - (sqt[:, None] + sqa[None, :])
    iota_f = lax.broadcasted_iota(jnp.int32, (TR, N), 1).astype(jnp.float32)
    cols = []
    for _ in range(K):
        m = jnp.max(x, axis=1, keepdims=True)
        eq = x == m
        amin = jnp.min(jnp.where(eq, iota_f, jnp.float32(N)),
                       axis=1, keepdims=True)
        cols.append(amin)
        x = jnp.where(eq, -jnp.inf, x)
    idx_ref[0] = (jnp.concatenate(cols, axis=1).astype(jnp.int32)
                  + b * N)


def _knn_topk_proj(coords, x6, w, bias, bg):
    mg = bg * N
    nt = N // TR
    cp = jnp.pad(coords, ((0, 0), (0, 0), (0, 5)))
    ct = jnp.transpose(cp, (0, 2, 1))
    return pl.pallas_call(
        _topk_body,
        grid=(bg, nt),
        in_specs=[
            pl.BlockSpec((1, TR, 8), lambda b, i: (b, i, 0)),
            pl.BlockSpec((1, 8, N), lambda b, i: (b, 0, 0)),
            pl.BlockSpec((TR, 8), lambda b, i: (b * (N // TR) + i, 0)),
            pl.BlockSpec((8, 192), lambda b, i: (0, 0)),
            pl.BlockSpec((1, 192), lambda b, i: (0, 0)),
        ],
        out_specs=[
            pl.BlockSpec((1, TR, K), lambda b, i: (b, i, 0)),
            pl.BlockSpec((TR, 64), lambda b, i: (b * (N // TR) + i, 0)),
            pl.BlockSpec((TR, 128), lambda b, i: (b * (N // TR) + i, 0)),
        ],
        out_shape=[
            jax.ShapeDtypeStruct((bg, N, K), jnp.int32),
            jax.ShapeDtypeStruct((mg, 64), jnp.float32),
            jax.ShapeDtypeStruct((mg, 128), jnp.float32),
        ],
    )(cp, ct, x6, w, bias[None, :])


def _relu_proj_body(dp, din, x_ref, m_ref, w_ref, b_ref, p2_ref, q2_ref):
    x1 = jnp.maximum(x_ref[...] + m_ref[:, :din], 0.0)
    h = jnp.dot(x1, w_ref[...],
                preferred_element_type=jnp.float32) + b_ref[...]
    p2_ref[...] = h[:, :dp]
    q2_ref[...] = h[:, dp:]


def _relu_proj(p, m, w, bias, dp, dq):
    mg, din = p.shape
    return pl.pallas_call(
        functools.partial(_relu_proj_body, dp, din),
        grid=(mg // 2048,),
        in_specs=[
            pl.BlockSpec((2048, din), lambda i: (i, 0)),
            pl.BlockSpec((2048, m.shape[1]), lambda i: (i, 0)),
            pl.BlockSpec(w.shape, lambda i: (0, 0)),
            pl.BlockSpec((1, dp + dq), lambda i: (0, 0)),
        ],
        out_specs=[
            pl.BlockSpec((2048, dp), lambda i: (i, 0)),
            pl.BlockSpec((2048, dq), lambda i: (i, 0)),
        ],
        out_shape=[
            jax.ShapeDtypeStruct((mg, dp), jnp.float32),
            jax.ShapeDtypeStruct((mg, dq), jnp.float32),
        ],
    )(p, m, w, bias[None, :])


def _gmax_body(c, nblk, ppw, mg, q_hbm, idx_hbm, out_hbm, idx_v, rows_v,
               out_v, shared, *sems):
    sid = lax.axis_index("s")
    wid = sid * 2 + lax.axis_index("c")
    pltpu.sync_copy(idx_hbm.at[wid], idx_v)
    stripe = mg // 16
    pltpu.sync_copy(q_hbm.at[pl.ds(sid * stripe, stripe)],
                    shared.at[pl.ds(sid * stripe, stripe)])
    plsc.subcore_barrier()
    nch = ppw // GCH

    def start(j, b):
        pltpu.async_copy(shared.at[idx_v.at[j]], rows_v.at[b], sems[b])

    for b in range(NBUF):
        start(b, b)

    def step(i, carry):
        j0 = i * NBUF
        for b in range(NBUF):
            j = j0 + b
            pltpu.make_async_copy(
                shared.at[idx_v.at[j]], rows_v.at[b], sems[b]).wait()
            ob = out_v.at[pl.ds(j * GCH, GCH)]
            for p in range(GCH):
                for cb in range(nblk):
                    sl = pl.ds(cb * 16, 16)
                    acc = rows_v[b, p * K, sl]
                    for r in range(1, K):
                        acc = jnp.maximum(acc, rows_v[b, p * K + r, sl])
                    ob[p, sl] = acc
            nxt = j + NBUF

            @pl.when(nxt < nch)
            def _():
                start(nxt, b)
        return carry

    lax.fori_loop(0, nch // NBUF, step, 0)
    pltpu.sync_copy(out_v, out_hbm.at[pl.ds(wid * ppw, ppw)])


def _gather_max(q, idx_w, c, nblk):
    mg = q.shape[0]
    ppw = mg // NW
    body = functools.partial(_gmax_body, c, nblk, ppw, mg)
    return pl.kernel(
        body,
        out_type=jax.ShapeDtypeStruct((mg, c), jnp.float32),
        mesh=plsc.VectorSubcoreMesh(core_axis_name="c", subcore_axis_name="s"),
        scratch_types=[
            pltpu.VMEM((ppw * K // 128, 128), jnp.int32),
            pltpu.VMEM((NBUF, 128, c), jnp.float32),
            pltpu.VMEM((ppw, c), jnp.float32),
            pltpu.VMEM_SHARED((mg, c), jnp.float32),
        ] + [pltpu.SemaphoreType.DMA] * NBUF,
    )(q, idx_w)


def _final_body(p_ref, m_ref, wf_ref, bf_ref, out_ref):
    x2 = jnp.maximum(p_ref[0] + m_ref[0], 0.0)
    pooled = jnp.max(x2, axis=0, keepdims=True)
    out_ref[0] = jnp.dot(pooled, wf_ref[...],
                         preferred_element_type=jnp.float32) + bf_ref[...]


def _final(p2, m2, wf, bf):
    bg = p2.shape[0] // N
    return pl.pallas_call(
        _final_body,
        grid=(bg,),
        in_specs=[
            pl.BlockSpec((1, N, 128), lambda b: (b, 0, 0)),
            pl.BlockSpec((1, N, 128), lambda b: (b, 0, 0)),
            pl.BlockSpec((128, 128), lambda b: (0, 0)),
            pl.BlockSpec((1, 128), lambda b: (0, 0)),
        ],
        out_specs=pl.BlockSpec((1, 1, 128), lambda b: (b, 0, 0)),
        out_shape=jax.ShapeDtypeStruct((bg, 1, 128), jnp.float32),
    )(p2.reshape(bg, N, 128), m2.reshape(bg, N, 128), wf, bf[None, :]
      ).reshape(bg, 128)


def kernel(feats, coords, W1, b1, W2, b2, Wf, bf):
    w1 = jnp.concatenate(
        [W1[:6] - W1[6:], W1[6:], jnp.zeros((6, 64), jnp.float32)], axis=1)
    w1 = jnp.pad(w1, ((0, 2), (0, 0)))
    bc1 = jnp.concatenate([b1, jnp.zeros((128,), jnp.float32)])
    w2 = jnp.concatenate([W2[:64] - W2[64:], W2[64:]], axis=1)
    bc2 = jnp.concatenate([b2, jnp.zeros((128,), jnp.float32)])

    outs = []
    for g in range(B // BG):
        s0 = g * BG
        mg = BG * N
        coords_g = lax.slice_in_dim(coords, s0, s0 + BG, axis=0)
        feats_g = lax.slice_in_dim(feats, s0, s0 + BG, axis=0)
        x6 = jnp.pad(feats_g.reshape(mg, 6), ((0, 0), (0, 2)))
        idx, p1, q1 = _knn_topk_proj(coords_g, x6, w1, bc1, BG)
        idx_w = idx.reshape(NW, mg * K // NW // 128, 128)
        m1 = _gather_max(q1, idx_w, 128, 4)
        p2, q2 = _relu_proj(p1, m1, w2, bc2, 128, 128)
        m2 = _gather_max(q2, idx_w, 128, 8)
        outs.append(_final(p2, m2, Wf, bf))
    return jnp.concatenate(outs, axis=0)


# speedup vs baseline: 1.0587x; 1.0587x over previous
import functools

import jax
import jax.numpy as jnp
from jax import lax
from jax.experimental import pallas as pl
from jax.experimental.pallas import tpu as pltpu
from jax.experimental.pallas import tpu_sc as plsc

B, N, K = 8, 2048, 16
M = B * N
TR = 512
NW = 32
GCH = 128 // K
BG = 8
NBUF = 2


def _topk_body(ct_ref, ca_ref, x_ref, w_ref, b_ref, idx_ref, p_ref, q_ref):
    b = pl.program_id(0)
    h = jnp.dot(x_ref[...], w_ref[...],
                preferred_element_type=jnp.float32) + b_ref[...]
    p_ref[...] = h[:, :64]
    q_ref[...] = h[:, 64:]
    ct = ct_ref[0]
    ca = ca_ref[0]
    dot = jnp.dot(ct, ca, preferred_element_type=jnp.float32)
    sqt = jnp.sum(ct * ct, axis=1)
    sqa = jnp.sum(ca * ca, axis=0)
    x = 2.0 * dot - (sqt[:, None] + sqa[None, :])
    iota_f = lax.broadcasted_iota(jnp.int32, (TR, N), 1).astype(jnp.float32)
    cols = []
    for _ in range(K):
        m = jnp.max(x, axis=1, keepdims=True)
        eq = x == m
        amin = jnp.min(jnp.where(eq, iota_f, jnp.float32(N)),
                       axis=1, keepdims=True)
        cols.append(amin)
        x = jnp.where(eq, -jnp.inf, x)
    idx_ref[0] = (jnp.concatenate(cols, axis=1).astype(jnp.int32)
                  + b * N)


def _knn_topk_proj(coords, x6, w, bias, bg):
    mg = bg * N
    nt = N // TR
    cp = jnp.pad(coords, ((0, 0), (0, 0), (0, 5)))
    ct = jnp.transpose(cp, (0, 2, 1))
    return pl.pallas_call(
        _topk_body,
        grid=(bg, nt),
        in_specs=[
            pl.BlockSpec((1, TR, 8), lambda b, i: (b, i, 0)),
            pl.BlockSpec((1, 8, N), lambda b, i: (b, 0, 0)),
            pl.BlockSpec((TR, 8), lambda b, i: (b * (N // TR) + i, 0)),
            pl.BlockSpec((8, 192), lambda b, i: (0, 0)),
            pl.BlockSpec((1, 192), lambda b, i: (0, 0)),
        ],
        out_specs=[
            pl.BlockSpec((1, TR, K), lambda b, i: (b, i, 0)),
            pl.BlockSpec((TR, 64), lambda b, i: (b * (N // TR) + i, 0)),
            pl.BlockSpec((TR, 128), lambda b, i: (b * (N // TR) + i, 0)),
        ],
        out_shape=[
            jax.ShapeDtypeStruct((bg, N, K), jnp.int32),
            jax.ShapeDtypeStruct((mg, 64), jnp.float32),
            jax.ShapeDtypeStruct((mg, 128), jnp.float32),
        ],
    )(cp, ct, x6, w, bias[None, :])


def _relu_proj_body(dp, din, x_ref, m_ref, w_ref, b_ref, p2_ref, q2_ref):
    x1 = jnp.maximum(x_ref[...] + m_ref[:, :din], 0.0)
    h = jnp.dot(x1, w_ref[...],
                preferred_element_type=jnp.float32) + b_ref[...]
    p2_ref[...] = h[:, :dp]
    q2_ref[...] = h[:, dp:]


def _relu_proj(p, m, w, bias, dp, dq):
    mg, din = p.shape
    return pl.pallas_call(
        functools.partial(_relu_proj_body, dp, din),
        grid=(mg // 2048,),
        in_specs=[
            pl.BlockSpec((2048, din), lambda i: (i, 0)),
            pl.BlockSpec((2048, m.shape[1]), lambda i: (i, 0)),
            pl.BlockSpec(w.shape, lambda i: (0, 0)),
            pl.BlockSpec((1, dp + dq), lambda i: (0, 0)),
        ],
        out_specs=[
            pl.BlockSpec((2048, dp), lambda i: (i, 0)),
            pl.BlockSpec((2048, dq), lambda i: (i, 0)),
        ],
        out_shape=[
            jax.ShapeDtypeStruct((mg, dp), jnp.float32),
            jax.ShapeDtypeStruct((mg, dq), jnp.float32),
        ],
    )(p, m, w, bias[None, :])


def _gmax_body(c, nblk, ppw, mg, stage, q_hbm, idx_hbm, out_hbm, idx_v,
               rows_v, out_v, shared, *sems):
    sid = lax.axis_index("s")
    wid = sid * 2 + lax.axis_index("c")
    pltpu.sync_copy(idx_hbm.at[wid], idx_v)
    if stage:
        stripe = mg // 16
        pltpu.sync_copy(q_hbm.at[pl.ds(sid * stripe, stripe)],
                        shared.at[pl.ds(sid * stripe, stripe)])
        plsc.subcore_barrier()
        table = shared
    else:
        table = q_hbm
    nch = ppw // GCH

    def start(j, b):
        pltpu.async_copy(table.at[idx_v.at[j]], rows_v.at[b], sems[b])

    for b in range(NBUF):
        start(b, b)

    def step(i, carry):
        j0 = i * NBUF
        for b in range(NBUF):
            j = j0 + b
            pltpu.make_async_copy(
                table.at[idx_v.at[j]], rows_v.at[b], sems[b]).wait()
            ob = out_v.at[pl.ds(j * GCH, GCH)]
            for p in range(GCH):
                for cb in range(nblk):
                    sl = pl.ds(cb * 16, 16)
                    acc = rows_v[b, p * K, sl]
                    for r in range(1, K):
                        acc = jnp.maximum(acc, rows_v[b, p * K + r, sl])
                    ob[p, sl] = acc
            nxt = j + NBUF

            @pl.when(nxt < nch)
            def _():
                start(nxt, b)
        return carry

    lax.fori_loop(0, nch // NBUF, step, 0)
    pltpu.sync_copy(out_v, out_hbm.at[pl.ds(wid * ppw, ppw)])


def _gather_max(q, idx_w, c, nblk):
    mg = q.shape[0]
    ppw = mg // NW
    stage = mg * c * 4 <= (2 << 20)
    body = functools.partial(_gmax_body, c, nblk, ppw, mg, stage)
    return pl.kernel(
        body,
        out_type=jax.ShapeDtypeStruct((mg, c), jnp.float32),
        mesh=plsc.VectorSubcoreMesh(core_axis_name="c", subcore_axis_name="s"),
        scratch_types=[
            pltpu.VMEM((ppw * K // 128, 128), jnp.int32),
            pltpu.VMEM((NBUF, 128, c), jnp.float32),
            pltpu.VMEM((ppw, c), jnp.float32),
            pltpu.VMEM_SHARED((mg if stage else 8, c), jnp.float32),
        ] + [pltpu.SemaphoreType.DMA] * NBUF,
    )(q, idx_w)


def _final_body(p_ref, m_ref, wf_ref, bf_ref, out_ref):
    x2 = jnp.maximum(p_ref[0] + m_ref[0], 0.0)
    pooled = jnp.max(x2, axis=0, keepdims=True)
    out_ref[0] = jnp.dot(pooled, wf_ref[...],
                         preferred_element_type=jnp.float32) + bf_ref[...]


def _final(p2, m2, wf, bf):
    bg = p2.shape[0] // N
    return pl.pallas_call(
        _final_body,
        grid=(bg,),
        in_specs=[
            pl.BlockSpec((1, N, 128), lambda b: (b, 0, 0)),
            pl.BlockSpec((1, N, 128), lambda b: (b, 0, 0)),
            pl.BlockSpec((128, 128), lambda b: (0, 0)),
            pl.BlockSpec((1, 128), lambda b: (0, 0)),
        ],
        out_specs=pl.BlockSpec((1, 1, 128), lambda b: (b, 0, 0)),
        out_shape=jax.ShapeDtypeStruct((bg, 1, 128), jnp.float32),
    )(p2.reshape(bg, N, 128), m2.reshape(bg, N, 128), wf, bf[None, :]
      ).reshape(bg, 128)


def kernel(feats, coords, W1, b1, W2, b2, Wf, bf):
    w1 = jnp.concatenate(
        [W1[:6] - W1[6:], W1[6:], jnp.zeros((6, 64), jnp.float32)], axis=1)
    w1 = jnp.pad(w1, ((0, 2), (0, 0)))
    bc1 = jnp.concatenate([b1, jnp.zeros((128,), jnp.float32)])
    w2 = jnp.concatenate([W2[:64] - W2[64:], W2[64:]], axis=1)
    bc2 = jnp.concatenate([b2, jnp.zeros((128,), jnp.float32)])

    outs = []
    for g in range(B // BG):
        s0 = g * BG
        mg = BG * N
        coords_g = lax.slice_in_dim(coords, s0, s0 + BG, axis=0)
        feats_g = lax.slice_in_dim(feats, s0, s0 + BG, axis=0)
        x6 = jnp.pad(feats_g.reshape(mg, 6), ((0, 0), (0, 2)))
        idx, p1, q1 = _knn_topk_proj(coords_g, x6, w1, bc1, BG)
        idx_w = idx.reshape(NW, mg * K // NW // 128, 128)
        m1 = _gather_max(q1, idx_w, 128, 4)
        p2, q2 = _relu_proj(p1, m1, w2, bc2, 128, 128)
        m2 = _gather_max(q2, idx_w, 128, 8)
        outs.append(_final(p2, m2, Wf, bf))
    return jnp.concatenate(outs, axis=0)


# speedup vs baseline: 1.1124x; 1.0507x over previous
import functools

import jax
import jax.numpy as jnp
from jax import lax
from jax.experimental import pallas as pl
from jax.experimental.pallas import tpu as pltpu
from jax.experimental.pallas import tpu_sc as plsc

B, N, K = 8, 2048, 16
M = B * N
TR = 512
NW = 32
GCH = 128 // K
BG = 2
NBUF = 2


def _topk_body(ct_ref, ca_ref, x_ref, w_ref, b_ref, idx_ref, p_ref, q_ref):
    b = pl.program_id(0)
    h = jnp.dot(x_ref[...], w_ref[...],
                preferred_element_type=jnp.float32) + b_ref[...]
    p_ref[...] = h[:, :64]
    q_ref[...] = h[:, 64:]
    ct = ct_ref[0]
    ca = ca_ref[0]
    dot = jnp.dot(ct, ca, preferred_element_type=jnp.float32)
    sqt = jnp.sum(ct * ct, axis=1)
    sqa = jnp.sum(ca * ca, axis=0)
    x = 2.0 * dot - (sqt[:, None] + sqa[None, :])
    iota_f = lax.broadcasted_iota(jnp.int32, (TR, N), 1).astype(jnp.float32)
    cols = []
    for _ in range(K):
        m = jnp.max(x, axis=1, keepdims=True)
        eq = x == m
        amin = jnp.min(jnp.where(eq, iota_f, jnp.float32(N)),
                       axis=1, keepdims=True)
        cols.append(amin)
        x = jnp.where(eq, -jnp.inf, x)
    idx_ref[0] = (jnp.concatenate(cols, axis=1).astype(jnp.int32)
                  + b * N)


def _knn_topk_proj(coords, x6, w, bias, bg):
    mg = bg * N
    nt = N // TR
    cp = jnp.pad(coords, ((0, 0), (0, 0), (0, 5)))
    ct = jnp.transpose(cp, (0, 2, 1))
    return pl.pallas_call(
        _topk_body,
        grid=(bg, nt),
        in_specs=[
            pl.BlockSpec((1, TR, 8), lambda b, i: (b, i, 0)),
            pl.BlockSpec((1, 8, N), lambda b, i: (b, 0, 0)),
            pl.BlockSpec((TR, 8), lambda b, i: (b * (N // TR) + i, 0)),
            pl.BlockSpec((8, 192), lambda b, i: (0, 0)),
            pl.BlockSpec((1, 192), lambda b, i: (0, 0)),
        ],
        out_specs=[
            pl.BlockSpec((1, TR, K), lambda b, i: (b, i, 0)),
            pl.BlockSpec((TR, 64), lambda b, i: (b * (N // TR) + i, 0)),
            pl.BlockSpec((TR, 128), lambda b, i: (b * (N // TR) + i, 0)),
        ],
        out_shape=[
            jax.ShapeDtypeStruct((bg, N, K), jnp.int32),
            jax.ShapeDtypeStruct((mg, 64), jnp.float32),
            jax.ShapeDtypeStruct((mg, 128), jnp.float32),
        ],
    )(cp, ct, x6, w, bias[None, :])


def _relu_proj_body(dp, din, x_ref, m_ref, w_ref, b_ref, p2_ref, q2_ref):
    x1 = jnp.maximum(x_ref[...] + m_ref[:, :din], 0.0)
    h = jnp.dot(x1, w_ref[...],
                preferred_element_type=jnp.float32) + b_ref[...]
    p2_ref[...] = h[:, :dp]
    q2_ref[...] = h[:, dp:]


def _relu_proj(p, m, w, bias, dp, dq):
    mg, din = p.shape
    return pl.pallas_call(
        functools.partial(_relu_proj_body, dp, din),
        grid=(mg // 2048,),
        in_specs=[
            pl.BlockSpec((2048, din), lambda i: (i, 0)),
            pl.BlockSpec((2048, m.shape[1]), lambda i: (i, 0)),
            pl.BlockSpec(w.shape, lambda i: (0, 0)),
            pl.BlockSpec((1, dp + dq), lambda i: (0, 0)),
        ],
        out_specs=[
            pl.BlockSpec((2048, dp), lambda i: (i, 0)),
            pl.BlockSpec((2048, dq), lambda i: (i, 0)),
        ],
        out_shape=[
            jax.ShapeDtypeStruct((mg, dp), jnp.float32),
            jax.ShapeDtypeStruct((mg, dq), jnp.float32),
        ],
    )(p, m, w, bias[None, :])


def _gmax_body(c, nblk, ppw, mg, stage, q_hbm, idx_hbm, out_hbm, idx_v,
               rows_v, out_v, shared, *sems):
    sid = lax.axis_index("s")
    wid = sid * 2 + lax.axis_index("c")
    pltpu.sync_copy(idx_hbm.at[wid], idx_v)
    if stage:
        stripe = mg // 16
        pltpu.sync_copy(q_hbm.at[pl.ds(sid * stripe, stripe)],
                        shared.at[pl.ds(sid * stripe, stripe)])
        plsc.subcore_barrier()
        table = shared
    else:
        table = q_hbm
    nch = ppw // GCH

    def start(j, b):
        pltpu.async_copy(table.at[idx_v.at[j]], rows_v.at[b], sems[b])

    for b in range(NBUF):
        start(b, b)

    def step(i, carry):
        j0 = i * NBUF
        for b in range(NBUF):
            j = j0 + b
            pltpu.make_async_copy(
                table.at[idx_v.at[j]], rows_v.at[b], sems[b]).wait()
            ob = out_v.at[pl.ds(j * GCH, GCH)]
            for p in range(GCH):
                for cb in range(nblk):
                    sl = pl.ds(cb * 16, 16)
                    acc = rows_v[b, p * K, sl]
                    for r in range(1, K):
                        acc = jnp.maximum(acc, rows_v[b, p * K + r, sl])
                    ob[p, sl] = acc
            nxt = j + NBUF

            @pl.when(nxt < nch)
            def _():
                start(nxt, b)
        return carry

    lax.fori_loop(0, nch // NBUF, step, 0)
    pltpu.sync_copy(out_v, out_hbm.at[pl.ds(wid * ppw, ppw)])


def _gather_max(q, idx_w, c, nblk):
    mg = q.shape[0]
    ppw = mg // NW
    stage = mg * c * 4 <= (2 << 20)
    body = functools.partial(_gmax_body, c, nblk, ppw, mg, stage)
    return pl.kernel(
        body,
        out_type=jax.ShapeDtypeStruct((mg, c), jnp.float32),
        mesh=plsc.VectorSubcoreMesh(core_axis_name="c", subcore_axis_name="s"),
        scratch_types=[
            pltpu.VMEM((ppw * K // 128, 128), jnp.int32),
            pltpu.VMEM((NBUF, 128, c), jnp.float32),
            pltpu.VMEM((ppw, c), jnp.float32),
            pltpu.VMEM_SHARED((mg if stage else 8, c), jnp.float32),
        ] + [pltpu.SemaphoreType.DMA] * NBUF,
    )(q, idx_w)


def _final_body(p_ref, m_ref, wf_ref, bf_ref, out_ref):
    x2 = jnp.maximum(p_ref[0] + m_ref[0], 0.0)
    pooled = jnp.max(x2, axis=0, keepdims=True)
    out_ref[0] = jnp.dot(pooled, wf_ref[...],
                         preferred_element_type=jnp.float32) + bf_ref[...]


def _final(p2, m2, wf, bf):
    bg = p2.shape[0] // N
    return pl.pallas_call(
        _final_body,
        grid=(bg,),
        in_specs=[
            pl.BlockSpec((1, N, 128), lambda b: (b, 0, 0)),
            pl.BlockSpec((1, N, 128), lambda b: (b, 0, 0)),
            pl.BlockSpec((128, 128), lambda b: (0, 0)),
            pl.BlockSpec((1, 128), lambda b: (0, 0)),
        ],
        out_specs=pl.BlockSpec((1, 1, 128), lambda b: (b, 0, 0)),
        out_shape=jax.ShapeDtypeStruct((bg, 1, 128), jnp.float32),
    )(p2.reshape(bg, N, 128), m2.reshape(bg, N, 128), wf, bf[None, :]
      ).reshape(bg, 128)


def kernel(feats, coords, W1, b1, W2, b2, Wf, bf):
    w1 = jnp.concatenate(
        [W1[:6] - W1[6:], W1[6:], jnp.zeros((6, 64), jnp.float32)], axis=1)
    w1 = jnp.pad(w1, ((0, 2), (0, 0)))
    bc1 = jnp.concatenate([b1, jnp.zeros((128,), jnp.float32)])
    w2 = jnp.concatenate([W2[:64] - W2[64:], W2[64:]], axis=1)
    bc2 = jnp.concatenate([b2, jnp.zeros((128,), jnp.float32)])

    outs = []
    for g in range(B // BG):
        s0 = g * BG
        mg = BG * N
        coords_g = lax.slice_in_dim(coords, s0, s0 + BG, axis=0)
        feats_g = lax.slice_in_dim(feats, s0, s0 + BG, axis=0)
        x6 = jnp.pad(feats_g.reshape(mg, 6), ((0, 0), (0, 2)))
        idx, p1, q1 = _knn_topk_proj(coords_g, x6, w1, bc1, BG)
        idx_w = idx.reshape(NW, mg * K // NW // 128, 128)
        m1 = _gather_max(q1, idx_w, 128, 4)
        p2, q2 = _relu_proj(p1, m1, w2, bc2, 128, 128)
        m2 = _gather_max(q2, idx_w, 128, 8)
        outs.append(_final(p2, m2, Wf, bf))
    return jnp.concatenate(outs, axis=0)
